# baseline (speedup 1.0000x reference)
import functools

import jax
import jax.numpy as jnp
from jax import lax
from jax.experimental import pallas as pl
from jax.experimental.pallas import tpu as pltpu
from jax.experimental.pallas import tpu_sc as plsc

N_NODES = 10000
N_EDGES = 160000
N_SEL = 9
NC = 2
NS = 16
FH = 128


def _gbase_kernel(src_ref, sel_ref, o_ref):
    o_ref[...] = src_ref[...] * N_SEL + sel_ref[...]


def _gather_base(src, sel):
    out = pl.pallas_call(
        _gbase_kernel,
        out_shape=jax.ShapeDtypeStruct((1250, 128), jnp.int32),
    )(src.reshape(1250, 128), sel.reshape(1250, 128))
    return out.reshape(N_EDGES)


def _mm1_kernel(x_ref, w_ref, o_ref):
    o_ref[0] = jnp.dot(x_ref[...], w_ref[0],
                       preferred_element_type=jnp.float32)


def _matmul_halves(x, w_r, bm):
    m, k = x.shape
    n = w_r.shape[2]
    return pl.pallas_call(
        _mm1_kernel,
        grid=(2, m // bm),
        in_specs=[pl.BlockSpec((bm, k), lambda c, j: (j, 0)),
                  pl.BlockSpec((1, k, n), lambda c, j: (c, 0, 0))],
        out_specs=pl.BlockSpec((1, bm, n), lambda c, j: (c, j, 0)),
        out_shape=jax.ShapeDtypeStruct((2, m, n), jnp.float32),
    )(x, w_r)


def _mm_cat_relu_kernel(a_ref, b_ref, ba_ref, bb_ref, wa_ref, wb_ref, o_ref):
    ha = jnp.maximum(a_ref[0] + ba_ref[...], 0.0)
    hb = jnp.maximum(b_ref[0] + bb_ref[...], 0.0)
    o_ref[...] = (jnp.dot(ha, wa_ref[...], preferred_element_type=jnp.float32)
                  + jnp.dot(hb, wb_ref[...],
                            preferred_element_type=jnp.float32))


def _matmul_cat_relu(agg, bias, w, bm):
    _, m, fi = agg.shape
    n = w.shape[1]
    ba, bb = bias[:fi].reshape(1, fi), bias[fi:].reshape(1, fi)
    wa, wb = w[:fi], w[fi:]
    return pl.pallas_call(
        _mm_cat_relu_kernel,
        grid=(m // bm,),
        in_specs=[pl.BlockSpec((1, bm, fi), lambda j: (0, j, 0)),
                  pl.BlockSpec((1, bm, fi), lambda j: (1, j, 0)),
                  pl.BlockSpec((1, fi), lambda j: (0, 0)),
                  pl.BlockSpec((1, fi), lambda j: (0, 0)),
                  pl.BlockSpec((fi, n), lambda j: (0, 0)),
                  pl.BlockSpec((fi, n), lambda j: (0, 0))],
        out_specs=pl.BlockSpec((bm, n), lambda j: (j, 0)),
        out_shape=jax.ShapeDtypeStruct((m, n), jnp.float32),
    )(agg, agg, ba, bb, wa, wb)


def _mm_sum_relu_kernel(a_ref, b_ref, bias_ref, w_ref, o_ref):
    h = jnp.maximum(a_ref[0] + b_ref[0] + bias_ref[...], 0.0)
    o_ref[...] = jnp.dot(h, w_ref[...], preferred_element_type=jnp.float32)


def _matmul_sum_relu(agg, bias, w, bm):
    _, m, fi = agg.shape
    n = w.shape[1]
    return pl.pallas_call(
        _mm_sum_relu_kernel,
        grid=(m // bm,),
        in_specs=[pl.BlockSpec((1, bm, fi), lambda j: (0, j, 0)),
                  pl.BlockSpec((1, bm, fi), lambda j: (1, j, 0)),
                  pl.BlockSpec((1, fi), lambda j: (0, 0)),
                  pl.BlockSpec((fi, n), lambda j: (0, 0))],
        out_specs=pl.BlockSpec((bm, n), lambda j: (j, 0)),
        out_shape=jax.ShapeDtypeStruct((m, n), jnp.float32),
    )(agg, agg, bias.reshape(1, fi), w)


def _gram_kernel(a_ref, b_ref, b3_ref, o_ref):
    h = (a_ref[...] + b_ref[...])[:, :32] + b3_ref[...]
    o_ref[...] = lax.dot_general(
        h, h, (((0,), (0,)), ((), ())),
        preferred_element_type=jnp.float32) * (1.0 / h.shape[0])


def _fc_kernel(f_ref, w_ref, b_ref, o_ref):
    o_ref[...] = lax.dot_general(
        f_ref[...], w_ref[...], (((1,), (1,)), ((), ())),
        preferred_element_type=jnp.float32) + b_ref[...]


def _head(agg3, b3, wfc, bfc):
    nfc = wfc.shape[0]
    gram = pl.pallas_call(
        _gram_kernel,
        out_shape=jax.ShapeDtypeStruct((32, 32), jnp.float32),
    )(agg3[0], agg3[1], b3.reshape(1, 32))
    flat = gram.reshape(1, 32 * 32)
    out = pl.pallas_call(
        _fc_kernel,
        out_shape=jax.ShapeDtypeStruct((1, nfc), jnp.float32),
    )(flat, wfc, bfc.reshape(1, nfc))
    return out.reshape(nfc)


def _splat16(vec, e):
    idx = jnp.full((16, 1), e, jnp.int32)
    return lax.gather(
        vec, idx,
        lax.GatherDimensionNumbers(offset_dims=(), collapsed_slice_dims=(0,),
                                   start_index_map=(0,)),
        (1,), mode=lax.GatherScatterMode.PROMISE_IN_BOUNDS)


def _sc_body(split_edges, n_chunks, chunk, nq_scale,
             table, gidx3_h, dst3_h, intp3_h, out_h,
             gidx_v, dst_v, intp_v, dstc_a, dstc_b, dstc_c,
             rows_a, rows_b, rows_c, acc_sh,
             gsem_a, gsem_b, gsem_c, ssem_a, ssem_b, ssem_c):
    c = lax.axis_index("c")
    s = lax.axis_index("s")
    if split_edges:
        shard = c * NS + s
        my_table = table
    else:
        shard = s
        my_table = table.at[c]
    nq = FH // 16
    nz_full = N_NODES // chunk
    nz_tail = N_NODES - nz_full * chunk
    z16f = jnp.zeros((16,), jnp.float32)

    pltpu.sync_copy(gidx3_h.at[shard], gidx_v)
    pltpu.sync_copy(dst3_h.at[shard], dst_v)
    pltpu.sync_copy(intp3_h.at[shard], intp_v)

    def _fill_dstc(j, dstc):
        for r in range(chunk // 16):
            dstc[pl.ds(r * 16, 16)] = dst_v[pl.ds(j * chunk + r * 16, 16)]

    def _zrow(i, _):
        for q in range(nq):
            rows_a[i, pl.ds(q * 16, 16)] = z16f
        return _
    lax.fori_loop(0, chunk, _zrow, 0)
    for i in range(pl.cdiv(nz_full + 1, NS)):
        b = s + i * NS
        @pl.when(b < nz_full)
        def _():
            pltpu.sync_copy(rows_a, acc_sh.at[pl.ds(b * chunk, chunk)])
        @pl.when(b == nz_full)
        def _():
            pltpu.sync_copy(rows_a.at[pl.ds(0, nz_tail)],
                            acc_sh.at[pl.ds(nz_full * chunk, nz_tail)])
    plsc.subcore_barrier()

    slots = ((dstc_a, rows_a, gsem_a, ssem_a),
             (dstc_b, rows_b, gsem_b, ssem_b),
             (dstc_c, rows_c, gsem_c, ssem_c))

    def _issue(j, rows, gsem):
        return pltpu.async_copy(my_table.at[gidx_v.at[pl.ds(j * chunk,
                                                            chunk)]],
                                rows, gsem)

    def _scale(j, rows):
        def _grp(g, _):
            iv = intp_v[pl.ds(j * chunk + g * 16, 16)]
            for e in range(16):
                splat = _splat16(iv, e)
                for q in range(nq_scale):
                    sl = pl.ds(q * 16, 16)
                    rows[g * 16 + e, sl] = rows[g * 16 + e, sl] * splat
            return _
        lax.fori_loop(0, chunk // 16, _grp, 0)

    _issue(0, rows_a, gsem_a)

    def _triple(t, carry):
        for k in range(3):
            j = 3 * t + k
            dstc_x, rows_x, gsem_x, _ssem_x = slots[k]
            dstc_y, rows_y, gsem_y, ssem_y = slots[(k + 1) % 3]
            @pl.when(j >= 2)
            def _drain():
                pltpu.make_async_copy(rows_y, acc_sh.at[dstc_y],
                                      ssem_y).wait()
            @pl.when(j + 1 < n_chunks)
            def _next():
                _issue(j + 1, rows_y, gsem_y)
            pltpu.make_async_copy(
                my_table.at[gidx_v.at[pl.ds(j * chunk, chunk)]], rows_x,
                gsem_x).wait()
            _scale(j, rows_x)
            _fill_dstc(j, dstc_x)
            pltpu.async_copy(rows_x, acc_sh.at[dstc_x], _ssem_x, add=True)
        return carry
    lax.fori_loop(0, n_chunks // 3, _triple, 0)
    pltpu.make_async_copy(rows_b, acc_sh.at[dstc_b], ssem_b).wait()
    pltpu.make_async_copy(rows_c, acc_sh.at[dstc_c], ssem_c).wait()
    plsc.subcore_barrier()

    for i in range(pl.cdiv(nz_full + 1, NS)):
        b = s + i * NS
        @pl.when(b < nz_full)
        def _():
            pltpu.sync_copy(acc_sh.at[pl.ds(b * chunk, chunk)],
                            out_h.at[c, pl.ds(b * chunk, chunk)])
        @pl.when(b == nz_full)
        def _():
            pltpu.sync_copy(acc_sh.at[pl.ds(nz_full * chunk, nz_tail)],
                            out_h.at[c, pl.ds(nz_full * chunk, nz_tail)])


def _shard_pad(arr, n_shards, e_pad):
    per = N_EDGES // n_shards
    return jnp.pad(arr.reshape(n_shards, per), ((0, 0), (0, e_pad - per)))


def _sc_aggregate(split_edges, chunk, nq_scale, table, gidx3, dst3, intp3):
    e_pad = gidx3.shape[1]
    n_chunks = e_pad // chunk
    mesh = plsc.VectorSubcoreMesh(core_axis_name="c", subcore_axis_name="s")
    kern = pl.kernel(
        functools.partial(_sc_body, split_edges, n_chunks, chunk, nq_scale),
        out_type=jax.ShapeDtypeStruct((NC, N_NODES, FH), jnp.float32),
        mesh=mesh,
        scratch_types=[
            pltpu.VMEM((e_pad,), jnp.int32),
            pltpu.VMEM((e_pad,), jnp.int32),
            pltpu.VMEM((e_pad,), jnp.float32),
            pltpu.VMEM((chunk,), jnp.int32),
            pltpu.VMEM((chunk,), jnp.int32),
            pltpu.VMEM((chunk,), jnp.int32),
            pltpu.VMEM((chunk, FH), jnp.float32),
            pltpu.VMEM((chunk, FH), jnp.float32),
            pltpu.VMEM((chunk, FH), jnp.float32),
            pltpu.VMEM_SHARED((N_NODES, FH), jnp.float32),
            pltpu.SemaphoreType.DMA,
            pltpu.SemaphoreType.DMA,
            pltpu.SemaphoreType.DMA,
            pltpu.SemaphoreType.DMA,
            pltpu.SemaphoreType.DMA,
            pltpu.SemaphoreType.DMA,
        ],
    )
    return kern(table, gidx3, dst3, intp3)


def kernel(x, edge_index, selections, interp_values,
           W1, b1, W2, b2, W3, b3, Wfc, bfc):
    src = edge_index[0]
    dst = edge_index[1]
    gbase = _gather_base(src, selections)
    gidx_a, dst_a, intp_a = (_shard_pad(a, NS, 210 * 48)
                             for a in (gbase, dst, interp_values))
    gidx_b, dst_b, intp_b = (_shard_pad(a, NC * NS, 81 * 64)
                             for a in (gbase, dst, interp_values))

    w1_r = (jnp.transpose(W1.reshape(N_SEL, 512, 2, 128), (2, 1, 0, 3))
            .reshape(2, 512, N_SEL * 128))
    w2_r = jnp.transpose(W2, (1, 0, 2)).reshape(256, N_SEL * 128)
    w3_r = (jnp.transpose(jnp.pad(W3, ((0, 0), (0, 0), (0, 96))), (1, 0, 2))
            .reshape(128, N_SEL * 128))

    y1 = _matmul_halves(x, w1_r, bm=1000)
    agg1 = _sc_aggregate(False, 48, 8, y1.reshape(2, N_NODES * N_SEL, FH),
                         gidx_a, dst_a, intp_a)
    y2 = _matmul_cat_relu(agg1, b1, w2_r, bm=1000)
    agg2 = _sc_aggregate(True, 64, 8, y2.reshape(N_NODES * N_SEL, FH),
                         gidx_b, dst_b, intp_b)
    y3 = _matmul_sum_relu(agg2, b2, w3_r, bm=1000)
    agg3 = _sc_aggregate(True, 64, 2, y3.reshape(N_NODES * N_SEL, FH),
                         gidx_b, dst_b, intp_b)
    return _head(agg3, b3, Wfc, bfc)


# speedup vs baseline: 1.1426x; 1.1426x over previous
import functools

import jax
import jax.numpy as jnp
from jax import lax
from jax.experimental import pallas as pl
from jax.experimental.pallas import tpu as pltpu
from jax.experimental.pallas import tpu_sc as plsc

N_NODES = 10000
N_EDGES = 160000
N_SEL = 9
NC = 2
NS = 16
FH = 128


def _gbase_kernel(src_ref, sel_ref, o_ref):
    o_ref[...] = src_ref[...] * N_SEL + sel_ref[...]


def _gather_base(src, sel):
    out = pl.pallas_call(
        _gbase_kernel,
        out_shape=jax.ShapeDtypeStruct((1250, 128), jnp.int32),
    )(src.reshape(1250, 128), sel.reshape(1250, 128))
    return out.reshape(N_EDGES)


def _mm1_kernel(x_ref, w_ref, o_ref):
    o_ref[0] = jnp.dot(x_ref[...], w_ref[0],
                       preferred_element_type=jnp.float32)


def _matmul_halves(x, w_r, bm):
    m, k = x.shape
    n = w_r.shape[2]
    return pl.pallas_call(
        _mm1_kernel,
        grid=(2, m // bm),
        in_specs=[pl.BlockSpec((bm, k), lambda c, j: (j, 0)),
                  pl.BlockSpec((1, k, n), lambda c, j: (c, 0, 0))],
        out_specs=pl.BlockSpec((1, bm, n), lambda c, j: (c, j, 0)),
        out_shape=jax.ShapeDtypeStruct((2, m, n), jnp.float32),
    )(x, w_r)


def _mm_cat_relu_kernel(a_ref, b_ref, ba_ref, bb_ref, wa_ref, wb_ref, o_ref):
    ha = jnp.maximum(a_ref[0] + ba_ref[...], 0.0)
    hb = jnp.maximum(b_ref[0] + bb_ref[...], 0.0)
    o_ref[...] = (jnp.dot(ha, wa_ref[...], preferred_element_type=jnp.float32)
                  + jnp.dot(hb, wb_ref[...],
                            preferred_element_type=jnp.float32))


def _matmul_cat_relu(agg, bias, w, bm):
    _, m, fi = agg.shape
    n = w.shape[1]
    ba, bb = bias[:fi].reshape(1, fi), bias[fi:].reshape(1, fi)
    wa, wb = w[:fi], w[fi:]
    return pl.pallas_call(
        _mm_cat_relu_kernel,
        grid=(m // bm,),
        in_specs=[pl.BlockSpec((1, bm, fi), lambda j: (0, j, 0)),
                  pl.BlockSpec((1, bm, fi), lambda j: (1, j, 0)),
                  pl.BlockSpec((1, fi), lambda j: (0, 0)),
                  pl.BlockSpec((1, fi), lambda j: (0, 0)),
                  pl.BlockSpec((fi, n), lambda j: (0, 0)),
                  pl.BlockSpec((fi, n), lambda j: (0, 0))],
        out_specs=pl.BlockSpec((bm, n), lambda j: (j, 0)),
        out_shape=jax.ShapeDtypeStruct((m, n), jnp.float32),
    )(agg, agg, ba, bb, wa, wb)


def _mm_sum_relu_kernel(a_ref, b_ref, bias_ref, w_ref, o_ref):
    h = jnp.maximum(a_ref[0] + b_ref[0] + bias_ref[...], 0.0)
    o_ref[...] = jnp.dot(h, w_ref[...], preferred_element_type=jnp.float32)


def _matmul_sum_relu(agg, bias, w, bm):
    _, m, fi = agg.shape
    n = w.shape[1]
    return pl.pallas_call(
        _mm_sum_relu_kernel,
        grid=(m // bm,),
        in_specs=[pl.BlockSpec((1, bm, fi), lambda j: (0, j, 0)),
                  pl.BlockSpec((1, bm, fi), lambda j: (1, j, 0)),
                  pl.BlockSpec((1, fi), lambda j: (0, 0)),
                  pl.BlockSpec((fi, n), lambda j: (0, 0))],
        out_specs=pl.BlockSpec((bm, n), lambda j: (j, 0)),
        out_shape=jax.ShapeDtypeStruct((m, n), jnp.float32),
    )(agg, agg, bias.reshape(1, fi), w)


def _gram_kernel(a_ref, b_ref, b3_ref, o_ref):
    h = (a_ref[...] + b_ref[...])[:, :32] + b3_ref[...]
    o_ref[...] = lax.dot_general(
        h, h, (((0,), (0,)), ((), ())),
        preferred_element_type=jnp.float32) * (1.0 / h.shape[0])


def _fc_kernel(f_ref, w_ref, b_ref, o_ref):
    o_ref[...] = lax.dot_general(
        f_ref[...], w_ref[...], (((1,), (1,)), ((), ())),
        preferred_element_type=jnp.float32) + b_ref[...]


def _head(agg3, b3, wfc, bfc):
    nfc = wfc.shape[0]
    gram = pl.pallas_call(
        _gram_kernel,
        out_shape=jax.ShapeDtypeStruct((32, 32), jnp.float32),
    )(agg3[0], agg3[1], b3.reshape(1, 32))
    flat = gram.reshape(1, 32 * 32)
    out = pl.pallas_call(
        _fc_kernel,
        out_shape=jax.ShapeDtypeStruct((1, nfc), jnp.float32),
    )(flat, wfc, bfc.reshape(1, nfc))
    return out.reshape(nfc)


def _splat16(vec, e):
    idx = jnp.full((16, 1), e, jnp.int32)
    return lax.gather(
        vec, idx,
        lax.GatherDimensionNumbers(offset_dims=(), collapsed_slice_dims=(0,),
                                   start_index_map=(0,)),
        (1,), mode=lax.GatherScatterMode.PROMISE_IN_BOUNDS)


def _sc_body(split_edges, n_chunks, chunk, nq_scale,
             table, gidx3_h, dst3_h, intp3_h, out_h,
             gidx_v, dst_v, intp_v, dstc_a, dstc_b, rows_a, rows_b, acc_sh,
             gsem_a, gsem_b):
    c = lax.axis_index("c")
    s = lax.axis_index("s")
    if split_edges:
        shard = c * NS + s
        my_table = table
    else:
        shard = s
        my_table = table.at[c]
    nq = FH // 16
    nz_full = N_NODES // chunk
    nz_tail = N_NODES - nz_full * chunk
    z16f = jnp.zeros((16,), jnp.float32)

    pltpu.async_copy(gidx3_h.at[shard], gidx_v, gsem_a)
    pltpu.async_copy(dst3_h.at[shard], dst_v, gsem_a)
    pltpu.async_copy(intp3_h.at[shard], intp_v, gsem_a)

    def _fill_dstc(j, dstc):
        for r in range(chunk // 16):
            dstc[pl.ds(r * 16, 16)] = dst_v[pl.ds(j * chunk + r * 16, 16)]

    def _zrow(i, _):
        for q in range(nq):
            rows_a[i, pl.ds(q * 16, 16)] = z16f
        return _
    lax.fori_loop(0, chunk, _zrow, 0)
    for i in range(pl.cdiv(nz_full + 1, NS)):
        b = s + i * NS
        @pl.when(b < nz_full)
        def _():
            pltpu.sync_copy(rows_a, acc_sh.at[pl.ds(b * chunk, chunk)])
        @pl.when(b == nz_full)
        def _():
            pltpu.sync_copy(rows_a.at[pl.ds(0, nz_tail)],
                            acc_sh.at[pl.ds(nz_full * chunk, nz_tail)])
    pltpu.make_async_copy(gidx3_h.at[shard], gidx_v, gsem_a).wait()
    pltpu.make_async_copy(dst3_h.at[shard], dst_v, gsem_a).wait()
    pltpu.make_async_copy(intp3_h.at[shard], intp_v, gsem_a).wait()
    plsc.subcore_barrier()

    def _issue(j, rows, gsem):
        return pltpu.async_copy(my_table.at[gidx_v.at[pl.ds(j * chunk,
                                                            chunk)]],
                                rows, gsem)

    def _scale(j, rows):
        def _grp(g, _):
            iv = intp_v[pl.ds(j * chunk + g * 16, 16)]
            for e in range(16):
                splat = _splat16(iv, e)
                for q in range(nq_scale):
                    sl = pl.ds(q * 16, 16)
                    rows[g * 16 + e, sl] = rows[g * 16 + e, sl] * splat
            return _
        lax.fori_loop(0, chunk // 16, _grp, 0)

    _issue(0, rows_a, gsem_a)

    def _pair(t, carry):
        j0 = 2 * t
        j1 = j0 + 1
        _issue(j1, rows_b, gsem_b)
        _fill_dstc(j0, dstc_a)
        pltpu.make_async_copy(
            my_table.at[gidx_v.at[pl.ds(j0 * chunk, chunk)]], rows_a,
            gsem_a).wait()
        _scale(j0, rows_a)
        pltpu.sync_copy(rows_a, acc_sh.at[dstc_a], add=True)
        @pl.when(j1 + 1 < n_chunks)
        def _issue_next():
            _issue(j1 + 1, rows_a, gsem_a)
        _fill_dstc(j1, dstc_b)
        pltpu.make_async_copy(
            my_table.at[gidx_v.at[pl.ds(j1 * chunk, chunk)]], rows_b,
            gsem_b).wait()
        _scale(j1, rows_b)
        pltpu.sync_copy(rows_b, acc_sh.at[dstc_b], add=True)
        return carry
    lax.fori_loop(0, n_chunks // 2, _pair, 0)
    plsc.subcore_barrier()

    for i in range(pl.cdiv(nz_full + 1, NS)):
        b = s + i * NS
        @pl.when(b < nz_full)
        def _():
            pltpu.sync_copy(acc_sh.at[pl.ds(b * chunk, chunk)],
                            out_h.at[c, pl.ds(b * chunk, chunk)])
        @pl.when(b == nz_full)
        def _():
            pltpu.sync_copy(acc_sh.at[pl.ds(nz_full * chunk, nz_tail)],
                            out_h.at[c, pl.ds(nz_full * chunk, nz_tail)])


def _shard_pad(arr, n_shards, e_pad):
    per = N_EDGES // n_shards
    return jnp.pad(arr.reshape(n_shards, per), ((0, 0), (0, e_pad - per)))


def _sc_aggregate(split_edges, chunk, nq_scale, table, gidx3, dst3, intp3):
    e_pad = gidx3.shape[1]
    n_chunks = e_pad // chunk
    mesh = plsc.VectorSubcoreMesh(core_axis_name="c", subcore_axis_name="s")
    kern = pl.kernel(
        functools.partial(_sc_body, split_edges, n_chunks, chunk, nq_scale),
        out_type=jax.ShapeDtypeStruct((NC, N_NODES, FH), jnp.float32),
        mesh=mesh,
        scratch_types=[
            pltpu.VMEM((e_pad,), jnp.int32),
            pltpu.VMEM((e_pad,), jnp.int32),
            pltpu.VMEM((e_pad,), jnp.float32),
            pltpu.VMEM((chunk,), jnp.int32),
            pltpu.VMEM((chunk,), jnp.int32),
            pltpu.VMEM((chunk, FH), jnp.float32),
            pltpu.VMEM((chunk, FH), jnp.float32),
            pltpu.VMEM_SHARED((N_NODES, FH), jnp.float32),
            pltpu.SemaphoreType.DMA,
            pltpu.SemaphoreType.DMA,
        ],
    )
    return kern(table, gidx3, dst3, intp3)


def kernel(x, edge_index, selections, interp_values,
           W1, b1, W2, b2, W3, b3, Wfc, bfc):
    src = edge_index[0]
    dst = edge_index[1]
    gbase = _gather_base(src, selections)
    gidx_a, dst_a, intp_a = (_shard_pad(a, NS, 158 * 64)
                             for a in (gbase, dst, interp_values))
    gidx_b, dst_b, intp_b = (_shard_pad(a, NC * NS, 40 * 128)
                             for a in (gbase, dst, interp_values))

    w1_r = (jnp.transpose(W1.reshape(N_SEL, 512, 2, 128), (2, 1, 0, 3))
            .reshape(2, 512, N_SEL * 128))
    w2_r = jnp.transpose(W2, (1, 0, 2)).reshape(256, N_SEL * 128)
    w3_r = (jnp.transpose(jnp.pad(W3, ((0, 0), (0, 0), (0, 96))), (1, 0, 2))
            .reshape(128, N_SEL * 128))

    y1 = _matmul_halves(x, w1_r, bm=1000)
    agg1 = _sc_aggregate(False, 64, 8, y1.reshape(2, N_NODES * N_SEL, FH),
                         gidx_a, dst_a, intp_a)
    y2 = _matmul_cat_relu(agg1, b1, w2_r, bm=1000)
    agg2 = _sc_aggregate(True, 128, 8, y2.reshape(N_NODES * N_SEL, FH),
                         gidx_b, dst_b, intp_b)
    y3 = _matmul_sum_relu(agg2, b2, w3_r, bm=1000)
    agg3 = _sc_aggregate(True, 128, 2, y3.reshape(N_NODES * N_SEL, FH),
                         gidx_b, dst_b, intp_b)
    return _head(agg3, b3, Wfc, bfc)


# speedup vs baseline: 1.1816x; 1.0341x over previous
import functools

import jax
import jax.numpy as jnp
from jax import lax
from jax.experimental import pallas as pl
from jax.experimental.pallas import tpu as pltpu
from jax.experimental.pallas import tpu_sc as plsc

N_NODES = 10000
N_EDGES = 160000
N_SEL = 9
NC = 2
NS = 16
FH = 128


def _gbase_kernel(src_ref, sel_ref, o_ref):
    o_ref[...] = src_ref[...] * N_SEL + sel_ref[...]


def _gather_base(src, sel):
    out = pl.pallas_call(
        _gbase_kernel,
        out_shape=jax.ShapeDtypeStruct((1250, 128), jnp.int32),
    )(src.reshape(1250, 128), sel.reshape(1250, 128))
    return out.reshape(N_EDGES)


def _mm1_kernel(x_ref, w_ref, o_ref):
    o_ref[0] = jnp.dot(x_ref[...], w_ref[0],
                       preferred_element_type=jnp.float32)


def _matmul_halves(x, w_r, bm):
    m, k = x.shape
    n = w_r.shape[2]
    return pl.pallas_call(
        _mm1_kernel,
        grid=(2, m // bm),
        in_specs=[pl.BlockSpec((bm, k), lambda c, j: (j, 0)),
                  pl.BlockSpec((1, k, n), lambda c, j: (c, 0, 0))],
        out_specs=pl.BlockSpec((1, bm, n), lambda c, j: (c, j, 0)),
        out_shape=jax.ShapeDtypeStruct((2, m, n), jnp.float32),
    )(x, w_r)


def _mm_cat_relu_kernel(a_ref, b_ref, ba_ref, bb_ref, wa_ref, wb_ref, o_ref):
    ha = jnp.maximum(a_ref[0] + ba_ref[...], 0.0)
    hb = jnp.maximum(b_ref[0] + bb_ref[...], 0.0)
    o_ref[...] = (jnp.dot(ha, wa_ref[...], preferred_element_type=jnp.float32)
                  + jnp.dot(hb, wb_ref[...],
                            preferred_element_type=jnp.float32))


def _matmul_cat_relu(agg, bias, w, bm):
    _, m, fi = agg.shape
    n = w.shape[1]
    ba, bb = bias[:fi].reshape(1, fi), bias[fi:].reshape(1, fi)
    wa, wb = w[:fi], w[fi:]
    return pl.pallas_call(
        _mm_cat_relu_kernel,
        grid=(m // bm,),
        in_specs=[pl.BlockSpec((1, bm, fi), lambda j: (0, j, 0)),
                  pl.BlockSpec((1, bm, fi), lambda j: (1, j, 0)),
                  pl.BlockSpec((1, fi), lambda j: (0, 0)),
                  pl.BlockSpec((1, fi), lambda j: (0, 0)),
                  pl.BlockSpec((fi, n), lambda j: (0, 0)),
                  pl.BlockSpec((fi, n), lambda j: (0, 0))],
        out_specs=pl.BlockSpec((bm, n), lambda j: (j, 0)),
        out_shape=jax.ShapeDtypeStruct((m, n), jnp.float32),
    )(agg, agg, ba, bb, wa, wb)


def _mm_sum_relu_kernel(a_ref, b_ref, bias_ref, w_ref, o_ref):
    h = jnp.maximum(a_ref[0] + b_ref[0] + bias_ref[...], 0.0)
    o_ref[...] = jnp.dot(h, w_ref[...], preferred_element_type=jnp.float32)


def _matmul_sum_relu(agg, bias, w, bm):
    _, m, fi = agg.shape
    n = w.shape[1]
    return pl.pallas_call(
        _mm_sum_relu_kernel,
        grid=(m // bm,),
        in_specs=[pl.BlockSpec((1, bm, fi), lambda j: (0, j, 0)),
                  pl.BlockSpec((1, bm, fi), lambda j: (1, j, 0)),
                  pl.BlockSpec((1, fi), lambda j: (0, 0)),
                  pl.BlockSpec((fi, n), lambda j: (0, 0))],
        out_specs=pl.BlockSpec((bm, n), lambda j: (j, 0)),
        out_shape=jax.ShapeDtypeStruct((m, n), jnp.float32),
    )(agg, agg, bias.reshape(1, fi), w)


def _gram_kernel(a_ref, b_ref, b3_ref, o_ref):
    h = (a_ref[...] + b_ref[...])[:, :32] + b3_ref[...]
    o_ref[...] = lax.dot_general(
        h, h, (((0,), (0,)), ((), ())),
        preferred_element_type=jnp.float32) * (1.0 / h.shape[0])


def _fc_kernel(f_ref, w_ref, b_ref, o_ref):
    o_ref[...] = lax.dot_general(
        f_ref[...], w_ref[...], (((1,), (1,)), ((), ())),
        preferred_element_type=jnp.float32) + b_ref[...]


def _head(agg3, b3, wfc, bfc):
    nfc = wfc.shape[0]
    gram = pl.pallas_call(
        _gram_kernel,
        out_shape=jax.ShapeDtypeStruct((32, 32), jnp.float32),
    )(agg3[0], agg3[1], b3.reshape(1, 32))
    flat = gram.reshape(1, 32 * 32)
    out = pl.pallas_call(
        _fc_kernel,
        out_shape=jax.ShapeDtypeStruct((1, nfc), jnp.float32),
    )(flat, wfc, bfc.reshape(1, nfc))
    return out.reshape(nfc)


def _splat16(vec, e):
    idx = jnp.full((16, 1), e, jnp.int32)
    return lax.gather(
        vec, idx,
        lax.GatherDimensionNumbers(offset_dims=(), collapsed_slice_dims=(0,),
                                   start_index_map=(0,)),
        (1,), mode=lax.GatherScatterMode.PROMISE_IN_BOUNDS)


def _sc_body(split_edges, n_chunks, chunk, nq_scale,
             table, gidx3_h, dst3_h, intp3_h, out_h,
             gidx_v, dst_v, intp_v, dstc_a, dstc_b, rows_a, rows_b, acc_sh,
             gsem_a, gsem_b):
    c = lax.axis_index("c")
    s = lax.axis_index("s")
    if split_edges:
        shard = c * NS + s
        my_table = table
    else:
        shard = s
        my_table = table.at[c]
    nq = FH // 16
    nz_full = N_NODES // chunk
    nz_tail = N_NODES - nz_full * chunk
    nz_blocks = nz_full + (1 if nz_tail else 0)
    z16f = jnp.zeros((16,), jnp.float32)

    pltpu.async_copy(gidx3_h.at[shard], gidx_v, gsem_a)
    pltpu.async_copy(dst3_h.at[shard], dst_v, gsem_a)
    pltpu.async_copy(intp3_h.at[shard], intp_v, gsem_a)

    def _fill_dstc(j, dstc):
        for r in range(chunk // 16):
            dstc[pl.ds(r * 16, 16)] = dst_v[pl.ds(j * chunk + r * 16, 16)]

    def _zrow(i, _):
        for q in range(nq):
            rows_a[i, pl.ds(q * 16, 16)] = z16f
        return _
    lax.fori_loop(0, chunk, _zrow, 0)
    for i in range(pl.cdiv(nz_blocks, NS)):
        b = s + i * NS
        @pl.when(b < nz_full)
        def _():
            pltpu.sync_copy(rows_a, acc_sh.at[pl.ds(b * chunk, chunk)])
        if nz_tail:
            @pl.when(b == nz_full)
            def _():
                pltpu.sync_copy(rows_a.at[pl.ds(0, nz_tail)],
                                acc_sh.at[pl.ds(nz_full * chunk, nz_tail)])
    pltpu.make_async_copy(gidx3_h.at[shard], gidx_v, gsem_a).wait()
    pltpu.make_async_copy(dst3_h.at[shard], dst_v, gsem_a).wait()
    pltpu.make_async_copy(intp3_h.at[shard], intp_v, gsem_a).wait()
    plsc.subcore_barrier()

    def _issue(j, rows, gsem):
        return pltpu.async_copy(my_table.at[gidx_v.at[pl.ds(j * chunk,
                                                            chunk)]],
                                rows, gsem)

    def _scale(j, rows):
        def _grp(g, _):
            iv = intp_v[pl.ds(j * chunk + g * 16, 16)]
            for e in range(16):
                splat = _splat16(iv, e)
                for q in range(nq_scale):
                    sl = pl.ds(q * 16, 16)
                    rows[g * 16 + e, sl] = rows[g * 16 + e, sl] * splat
            return _
        lax.fori_loop(0, chunk // 16, _grp, 0)

    _issue(0, rows_a, gsem_a)

    def _pair(t, carry):
        j0 = 2 * t
        j1 = j0 + 1
        _issue(j1, rows_b, gsem_b)
        _fill_dstc(j0, dstc_a)
        pltpu.make_async_copy(
            my_table.at[gidx_v.at[pl.ds(j0 * chunk, chunk)]], rows_a,
            gsem_a).wait()
        _scale(j0, rows_a)
        pltpu.sync_copy(rows_a, acc_sh.at[dstc_a], add=True)
        @pl.when(j1 + 1 < n_chunks)
        def _issue_next():
            _issue(j1 + 1, rows_a, gsem_a)
        _fill_dstc(j1, dstc_b)
        pltpu.make_async_copy(
            my_table.at[gidx_v.at[pl.ds(j1 * chunk, chunk)]], rows_b,
            gsem_b).wait()
        _scale(j1, rows_b)
        pltpu.sync_copy(rows_b, acc_sh.at[dstc_b], add=True)
        return carry
    lax.fori_loop(0, n_chunks // 2, _pair, 0)
    plsc.subcore_barrier()

    for i in range(pl.cdiv(nz_blocks, NS)):
        b = s + i * NS
        @pl.when(b < nz_full)
        def _():
            pltpu.sync_copy(acc_sh.at[pl.ds(b * chunk, chunk)],
                            out_h.at[c, pl.ds(b * chunk, chunk)])
        if nz_tail:
            @pl.when(b == nz_full)
            def _():
                pltpu.sync_copy(acc_sh.at[pl.ds(nz_full * chunk, nz_tail)],
                                out_h.at[c, pl.ds(nz_full * chunk, nz_tail)])


def _shard_pad(arr, n_shards, e_pad):
    per = N_EDGES // n_shards
    return jnp.pad(arr.reshape(n_shards, per), ((0, 0), (0, e_pad - per)))


def _sc_aggregate(split_edges, chunk, nq_scale, table, gidx3, dst3, intp3):
    e_pad = gidx3.shape[1]
    n_chunks = e_pad // chunk
    mesh = plsc.VectorSubcoreMesh(core_axis_name="c", subcore_axis_name="s")
    kern = pl.kernel(
        functools.partial(_sc_body, split_edges, n_chunks, chunk, nq_scale),
        out_type=jax.ShapeDtypeStruct((NC, N_NODES, FH), jnp.float32),
        mesh=mesh,
        scratch_types=[
            pltpu.VMEM((e_pad,), jnp.int32),
            pltpu.VMEM((e_pad,), jnp.int32),
            pltpu.VMEM((e_pad,), jnp.float32),
            pltpu.VMEM((chunk,), jnp.int32),
            pltpu.VMEM((chunk,), jnp.int32),
            pltpu.VMEM((chunk, FH), jnp.float32),
            pltpu.VMEM((chunk, FH), jnp.float32),
            pltpu.VMEM_SHARED((N_NODES, FH), jnp.float32),
            pltpu.SemaphoreType.DMA,
            pltpu.SemaphoreType.DMA,
        ],
    )
    return kern(table, gidx3, dst3, intp3)


def kernel(x, edge_index, selections, interp_values,
           W1, b1, W2, b2, W3, b3, Wfc, bfc):
    src = edge_index[0]
    dst = edge_index[1]
    gbase = _gather_base(src, selections)
    gidx_a, dst_a, intp_a = (_shard_pad(a, NS, 126 * 80)
                             for a in (gbase, dst, interp_values))
    gidx_b, dst_b, intp_b = (_shard_pad(a, NC * NS, 40 * 128)
                             for a in (gbase, dst, interp_values))

    w1_r = (jnp.transpose(W1.reshape(N_SEL, 512, 2, 128), (2, 1, 0, 3))
            .reshape(2, 512, N_SEL * 128))
    w2_r = jnp.transpose(W2, (1, 0, 2)).reshape(256, N_SEL * 128)
    w3_r = (jnp.transpose(jnp.pad(W3, ((0, 0), (0, 0), (0, 96))), (1, 0, 2))
            .reshape(128, N_SEL * 128))

    y1 = _matmul_halves(x, w1_r, bm=1000)
    agg1 = _sc_aggregate(False, 80, 8, y1.reshape(2, N_NODES * N_SEL, FH),
                         gidx_a, dst_a, intp_a)
    y2 = _matmul_cat_relu(agg1, b1, w2_r, bm=1000)
    agg2 = _sc_aggregate(True, 128, 8, y2.reshape(N_NODES * N_SEL, FH),
                         gidx_b, dst_b, intp_b)
    y3 = _matmul_sum_relu(agg2, b2, w3_r, bm=1000)
    agg3 = _sc_aggregate(True, 128, 2, y3.reshape(N_NODES * N_SEL, FH),
                         gidx_b, dst_b, intp_b)
    return _head(agg3, b3, Wfc, bfc)
